# baseline (speedup 1.0000x reference)
import jax
import jax.numpy as jnp
from jax import lax
from jax.experimental import pallas as pl
from jax.experimental.pallas import tpu as pltpu
from jax.experimental.pallas import tpu_sc as plsc

_PAD = 1
_NC = 2
_NS = 16
_NW = _NC * _NS
_LANES = 16
_NACC = 4
_CB = 32768


def _tr_body(wt_ref, out_ref):
    x = wt_ref[...]
    xt = jnp.transpose(x)
    out_ref[...] = jnp.concatenate(
        [xt[: _CB // 2], xt[_CB // 2:]], axis=1)


def _u_of(v):
    half = _CB // 2
    i, r = divmod(v, _CB)
    return i * _CB + 2 * (r % half) + (r // half)


def _linearize_table(word_embeddings):
    V, E = word_embeddings.shape
    nblk = -(-V // _CB)
    tr = pl.pallas_call(
        _tr_body,
        grid=(nblk,),
        in_specs=[pl.BlockSpec((E, _CB), lambda i: (0, i))],
        out_specs=pl.BlockSpec((_CB // 2, 2 * E), lambda i: (i, 0)),
        out_shape=jax.ShapeDtypeStruct((nblk * _CB // 2, 2 * E),
                                       jnp.float32),
    )
    lin2 = tr(word_embeddings.T)
    return lin2.reshape(-1, E)


def _cbow_body(sent_hbm, len_hbm, emb_hbm, out_hbm,
               idx_v, idx2_v, acc_v, out_v, len_v, cnt_v, rlen_v, emb1_v,
               sems):
    L, B = sent_hbm.shape
    _, E = emb_hbm.shape
    bpw = B // _NW
    nj = bpw // _LANES
    ev = E // _LANES
    wid = lax.axis_index("s") * _NC + lax.axis_index("c")
    base = wid * bpw

    pltpu.sync_copy(len_hbm.at[pl.ds(base, bpw)], len_v)
    pltpu.sync_copy(emb_hbm.at[pl.ds(_u_of(_PAD), 1)], emb1_v)
    pltpu.sync_copy(sent_hbm.at[:, pl.ds(base, bpw)], idx_v)

    half = _CB // 2

    def tbody(l, carry):
        out = []
        g = lax.div(l, 2)
        off = lax.rem(l, 2) * bpw
        for j in range(nj):
            v = idx_v[l, pl.ds(j * _LANES, _LANES)]
            out.append(carry[j] + (v == _PAD).astype(jnp.int32))
            r = lax.rem(v, _CB)
            u = (v - r) + 2 * lax.rem(r, half) + lax.div(r, half)
            idx2_v[g, pl.ds(off + j * _LANES, _LANES)] = u
        return tuple(out)
    cnt = lax.fori_loop(
        0, L, tbody,
        tuple(jnp.zeros((_LANES,), jnp.int32) for _ in range(nj)))

    G = L // 2
    for a in range(_NACC):
        pltpu.make_async_copy(
            emb_hbm.at[idx2_v.at[a]], acc_v.at[a], sems.at[a]).start()

    def step(g0, carry):
        for a in range(_NACC):
            g = g0 * _NACC + a
            pltpu.make_async_copy(
                emb_hbm.at[idx2_v.at[g - _NACC]], acc_v.at[a],
                sems.at[a]).wait()
            pltpu.make_async_copy(
                emb_hbm.at[idx2_v.at[g]], acc_v.at[a],
                sems.at[a]).start(add=True)
        return carry
    lax.fori_loop(1, G // _NACC, step, 0)

    for j in range(nj):
        ds = pl.ds(j * _LANES, _LANES)
        cnt_v[ds] = cnt[j].astype(jnp.float32)
        rlen_v[ds] = 1.0 / len_v[ds].astype(jnp.float32)

    for k in range(_NACC):
        g = G - _NACC + k
        pltpu.make_async_copy(
            emb_hbm.at[idx2_v.at[g]], acc_v.at[g % _NACC],
            sems.at[g % _NACC]).wait()

    def obody(b, carry):
        bb = jnp.full((_LANES,), b, jnp.int32)
        c = plsc.load_gather(cnt_v, [bb])
        r = plsc.load_gather(rlen_v, [bb])
        for e in range(ev):
            ds = pl.ds(e * _LANES, _LANES)
            tot = acc_v[0, b, ds] + acc_v[0, b + bpw, ds]
            for a in range(1, _NACC):
                tot = tot + (acc_v[a, b, ds] + acc_v[a, b + bpw, ds])
            out_v[b, ds] = (tot - c * emb1_v[0, ds]) * r
        return carry
    lax.fori_loop(0, bpw, obody, 0)

    pltpu.sync_copy(out_v, out_hbm.at[pl.ds(base, bpw)])


def kernel(input_sentence, lengths, word_embeddings):
    B, L = input_sentence.shape
    V, E = word_embeddings.shape
    bpw = B // _NW
    emb_lin = _linearize_table(word_embeddings)
    f = pl.kernel(
        _cbow_body,
        out_type=jax.ShapeDtypeStruct((B, E), jnp.float32),
        mesh=plsc.VectorSubcoreMesh(
            core_axis_name="c", subcore_axis_name="s",
            num_cores=_NC, num_subcores=_NS),
        scratch_types=[
            pltpu.VMEM((L, bpw), jnp.int32),
            pltpu.VMEM((L // 2, 2 * bpw), jnp.int32),
            pltpu.VMEM((_NACC, 2 * bpw, E), jnp.float32),
            pltpu.VMEM((bpw, E), jnp.float32),
            pltpu.VMEM((bpw,), jnp.int32),
            pltpu.VMEM((bpw,), jnp.float32),
            pltpu.VMEM((bpw,), jnp.float32),
            pltpu.VMEM((1, E), jnp.float32),
            pltpu.SemaphoreType.DMA((_NACC,)),
        ],
        compiler_params=pltpu.CompilerParams(
            use_tc_tiling_on_sc=False, needs_layout_passes=False),
    )
    return f(input_sentence.astype(jnp.int32).T, lengths.astype(jnp.int32),
             emb_lin)


# speedup vs baseline: 1.0641x; 1.0641x over previous
import jax
import jax.numpy as jnp
from jax import lax
from jax.experimental import pallas as pl
from jax.experimental.pallas import tpu as pltpu
from jax.experimental.pallas import tpu_sc as plsc

_PAD = 1
_NC = 2
_NS = 16
_NW = _NC * _NS
_LANES = 16
_NACC = 10
_CB = 32768


def _tr_body(wt_ref, out_ref):
    x = wt_ref[...]
    xt = jnp.transpose(x)
    out_ref[...] = jnp.concatenate(
        [xt[: _CB // 2], xt[_CB // 2:]], axis=1)


def _u_of(v):
    half = _CB // 2
    i, r = divmod(v, _CB)
    return i * _CB + 2 * (r % half) + (r // half)


def _linearize_table(word_embeddings):
    V, E = word_embeddings.shape
    nblk = -(-V // _CB)
    tr = pl.pallas_call(
        _tr_body,
        grid=(nblk,),
        in_specs=[pl.BlockSpec((E, _CB), lambda i: (0, i))],
        out_specs=pl.BlockSpec((_CB // 2, 2 * E), lambda i: (i, 0)),
        out_shape=jax.ShapeDtypeStruct((nblk * _CB // 2, 2 * E),
                                       jnp.float32),
    )
    lin2 = tr(word_embeddings.T)
    return lin2.reshape(-1, E)


def _cbow_body(sent_hbm, len_hbm, emb_hbm, out_hbm,
               idx_v, acc_v, out_v, len_v, cnt_v, rlen_v, emb1_v,
               sems):
    L, B = sent_hbm.shape
    _, E = emb_hbm.shape
    bpw = B // _NW
    nj = bpw // _LANES
    ev = E // _LANES
    wid = lax.axis_index("s") * _NC + lax.axis_index("c")
    base = wid * bpw

    pltpu.sync_copy(len_hbm.at[pl.ds(base, bpw)], len_v)
    pltpu.sync_copy(emb_hbm.at[pl.ds(_u_of(_PAD), 1)], emb1_v)
    pltpu.sync_copy(sent_hbm.at[:, pl.ds(base, bpw)], idx_v)

    half = _CB // 2

    def tbody(l, carry):
        out = []
        for j in range(nj):
            ds = pl.ds(j * _LANES, _LANES)
            v = idx_v[l, ds]
            out.append(carry[j] + (v == _PAD).astype(jnp.int32))
            r = lax.rem(v, _CB)
            u = (v - r) + 2 * lax.rem(r, half) + lax.div(r, half)
            idx_v[l, ds] = u
        return tuple(out)
    cnt = lax.fori_loop(
        0, L, tbody,
        tuple(jnp.zeros((_LANES,), jnp.int32) for _ in range(nj)))

    for a in range(_NACC):
        pltpu.make_async_copy(
            emb_hbm.at[idx_v.at[a]], acc_v.at[a], sems.at[a]).start()

    def step(g, carry):
        for a in range(_NACC):
            l = g * _NACC + a
            pltpu.make_async_copy(
                emb_hbm.at[idx_v.at[l - _NACC]], acc_v.at[a],
                sems.at[a]).wait()
            pltpu.make_async_copy(
                emb_hbm.at[idx_v.at[l]], acc_v.at[a],
                sems.at[a]).start(add=True)
        return carry
    lax.fori_loop(1, L // _NACC, step, 0)

    for j in range(nj):
        ds = pl.ds(j * _LANES, _LANES)
        cnt_v[ds] = cnt[j].astype(jnp.float32)
        rlen_v[ds] = 1.0 / len_v[ds].astype(jnp.float32)

    for k in range(_NACC):
        l = L - _NACC + k
        pltpu.make_async_copy(
            emb_hbm.at[idx_v.at[l]], acc_v.at[l % _NACC],
            sems.at[l % _NACC]).wait()

    def obody(b, carry):
        bb = jnp.full((_LANES,), b, jnp.int32)
        c = plsc.load_gather(cnt_v, [bb])
        r = plsc.load_gather(rlen_v, [bb])
        for e in range(ev):
            ds = pl.ds(e * _LANES, _LANES)
            tot = acc_v[0, b, ds]
            for a in range(1, _NACC):
                tot = tot + acc_v[a, b, ds]
            out_v[b, ds] = (tot - c * emb1_v[0, ds]) * r
        return carry
    lax.fori_loop(0, bpw, obody, 0)

    pltpu.sync_copy(out_v, out_hbm.at[pl.ds(base, bpw)])


def kernel(input_sentence, lengths, word_embeddings):
    B, L = input_sentence.shape
    V, E = word_embeddings.shape
    bpw = B // _NW
    emb_lin = _linearize_table(word_embeddings)
    f = pl.kernel(
        _cbow_body,
        out_type=jax.ShapeDtypeStruct((B, E), jnp.float32),
        mesh=plsc.VectorSubcoreMesh(
            core_axis_name="c", subcore_axis_name="s",
            num_cores=_NC, num_subcores=_NS),
        scratch_types=[
            pltpu.VMEM((L, bpw), jnp.int32),
            pltpu.VMEM((_NACC, bpw, E), jnp.float32),
            pltpu.VMEM((bpw, E), jnp.float32),
            pltpu.VMEM((bpw,), jnp.int32),
            pltpu.VMEM((bpw,), jnp.float32),
            pltpu.VMEM((bpw,), jnp.float32),
            pltpu.VMEM((1, E), jnp.float32),
            pltpu.SemaphoreType.DMA((_NACC,)),
        ],
        compiler_params=pltpu.CompilerParams(
            use_tc_tiling_on_sc=False, needs_layout_passes=False),
    )
    return f(input_sentence.astype(jnp.int32).T, lengths.astype(jnp.int32),
             emb_lin)
